# baseline (speedup 1.0000x reference)
import jax
import jax.numpy as jnp
from jax.experimental import pallas as pl

NUM_CODES = 1024
DIM = 64


def _vq_body(x_ref, cb_ref, zis_ref, zqs_ref):
    c, h, w = x_ref.shape[1:]
    zT = x_ref[0].reshape(c, h * w)
    cb = cb_ref[...]
    hw = zT.shape[1]

    se = jnp.sum(cb * cb, axis=1, keepdims=True)
    sz = jnp.sum(zT * zT, axis=0, keepdims=True)
    m2 = jax.lax.dot_general(cb + cb, zT, (((1,), (0,)), ((), ())),
                             preferred_element_type=jnp.float32)
    d = (sz + se) - m2

    dmin = jnp.min(d, axis=0, keepdims=True)
    codesf = jax.lax.broadcasted_iota(
        jnp.int32, (NUM_CODES, 1), 0).astype(jnp.float32)
    idxf = jnp.min(jnp.where(d == dmin, codesf, float(NUM_CODES)),
                   axis=0, keepdims=True)
    zis_ref[0] = idxf.astype(jnp.int32).reshape(h, w)

    onehot = (codesf == idxf).astype(jnp.float32)
    zq = jax.lax.dot_general(cb, onehot, (((0,), (0,)), ((), ())),
                             preferred_element_type=jnp.float32)
    zqs_ref[0] = zq.reshape(c, h, w)


def kernel(inputs, codebook):
    B, C, H, W = inputs.shape

    zis, zqs = pl.pallas_call(
        _vq_body,
        grid=(B,),
        in_specs=[
            pl.BlockSpec((1, C, H, W), lambda b: (b, 0, 0, 0)),
            pl.BlockSpec((NUM_CODES, DIM), lambda b: (0, 0)),
        ],
        out_specs=[
            pl.BlockSpec((1, H, W), lambda b: (b, 0, 0)),
            pl.BlockSpec((1, C, H, W), lambda b: (b, 0, 0, 0)),
        ],
        out_shape=[
            jax.ShapeDtypeStruct((B, H, W), jnp.int32),
            jax.ShapeDtypeStruct((B, C, H, W), jnp.float32),
        ],
    )(inputs, codebook)

    return zis, zqs


# speedup vs baseline: 1.5370x; 1.5370x over previous
import jax
import jax.numpy as jnp
from jax.experimental import pallas as pl

NUM_CODES = 1024
DIM = 64
PB = 2


def _vq_body(x_ref, cb_ref, zis_ref, zqs_ref):
    cb = cb_ref[...]
    cb2 = cb + cb
    se = jnp.sum(cb * cb, axis=1, keepdims=True)
    codesf = jax.lax.broadcasted_iota(
        jnp.int32, (NUM_CODES, 1), 0).astype(jnp.float32)

    for i in range(PB):
        zT = x_ref[i]
        hw = zT.shape[1]
        sz = jnp.sum(zT * zT, axis=0, keepdims=True)
        m2 = jax.lax.dot_general(cb2, zT, (((1,), (0,)), ((), ())),
                                 preferred_element_type=jnp.float32)
        d = (sz + se) - m2

        dmin = jnp.min(d, axis=0, keepdims=True)
        idxf = jnp.min(jnp.where(d == dmin, codesf, float(NUM_CODES)),
                       axis=0, keepdims=True)
        zis_ref[i] = idxf.astype(jnp.int32)

        onehot = (codesf == idxf).astype(jnp.float32)
        zq = jax.lax.dot_general(cb, onehot, (((0,), (0,)), ((), ())),
                                 preferred_element_type=jnp.float32)
        zqs_ref[i] = zq


def kernel(inputs, codebook):
    B, C, H, W = inputs.shape
    HW = H * W
    x = inputs.reshape(B, C, HW)

    zis3, zqs3 = pl.pallas_call(
        _vq_body,
        grid=(B // PB,),
        in_specs=[
            pl.BlockSpec((PB, C, HW), lambda b: (b, 0, 0)),
            pl.BlockSpec((NUM_CODES, DIM), lambda b: (0, 0)),
        ],
        out_specs=[
            pl.BlockSpec((PB, 1, HW), lambda b: (b, 0, 0)),
            pl.BlockSpec((PB, C, HW), lambda b: (b, 0, 0)),
        ],
        out_shape=[
            jax.ShapeDtypeStruct((B, 1, HW), jnp.int32),
            jax.ShapeDtypeStruct((B, C, HW), jnp.float32),
        ],
    )(x, codebook)

    return zis3.reshape(B, H, W), zqs3.reshape(B, C, H, W)


# speedup vs baseline: 1.6408x; 1.0675x over previous
import jax
import jax.numpy as jnp
from jax.experimental import pallas as pl

NUM_CODES = 1024
DIM = 64
PB = 2


def _vq_body(x_ref, cb_ref, zis_ref, zqs_ref):
    cb = cb_ref[...]
    cb2 = cb + cb
    se = jnp.sum(cb * cb, axis=1, keepdims=True)
    codesf = jax.lax.broadcasted_iota(
        jnp.int32, (NUM_CODES, 1), 0).astype(jnp.float32)

    for i in range(PB):
        zT = x_ref[i]
        sz = jnp.sum(zT * zT, axis=0, keepdims=True)
        m2 = jax.lax.dot_general(cb2, zT, (((1,), (0,)), ((), ())),
                                 preferred_element_type=jnp.float32)
        d = (sz + se) - m2

        dmin = jnp.min(d, axis=0, keepdims=True)
        hit = d == dmin
        idxf = jnp.min(jnp.where(hit, codesf, float(NUM_CODES)),
                       axis=0, keepdims=True)
        zis_ref[i] = idxf.astype(jnp.int32)

        onehot = hit.astype(jnp.float32)
        zq = jax.lax.dot_general(cb, onehot, (((0,), (0,)), ((), ())),
                                 preferred_element_type=jnp.float32)
        zqs_ref[i] = zq


def kernel(inputs, codebook):
    B, C, H, W = inputs.shape
    HW = H * W
    x = inputs.reshape(B, C, HW)

    zis3, zqs3 = pl.pallas_call(
        _vq_body,
        grid=(B // PB,),
        in_specs=[
            pl.BlockSpec((PB, C, HW), lambda b: (b, 0, 0)),
            pl.BlockSpec((NUM_CODES, DIM), lambda b: (0, 0)),
        ],
        out_specs=[
            pl.BlockSpec((PB, 1, HW), lambda b: (b, 0, 0)),
            pl.BlockSpec((PB, C, HW), lambda b: (b, 0, 0)),
        ],
        out_shape=[
            jax.ShapeDtypeStruct((B, 1, HW), jnp.int32),
            jax.ShapeDtypeStruct((B, C, HW), jnp.float32),
        ],
    )(x, codebook)

    return zis3.reshape(B, H, W), zqs3.reshape(B, C, H, W)
